# Initial kernel scaffold; baseline (speedup 1.0000x reference)
import jax
import jax.numpy as jnp
from jax.experimental import pallas as pl


def kernel(user_emb, item_emb, edge_index):
    raise NotImplementedError("write your pallas kernel here")


# speedup vs baseline: 6.6503x; 6.6503x over previous
import functools

import numpy as np

import jax
import jax.numpy as jnp
from jax import lax
from jax.experimental import pallas as pl
from jax.experimental.pallas import tpu as pltpu
from jax.experimental.pallas import tpu_sc as plsc

N_USERS = 25000
N_ITEMS = 25000
N_NODES = N_USERS + N_ITEMS
D = 64
E = 800000
HALF = 25000
P = 25088
NP = 2 * P
PAD = P - HALF
S_ROWS = P + 128
EB = 128
NBLK_E = E // EB
NBLK_N = P // 128
NBLK_S = S_ROWS // 128
MAGIC = np.int32(0x5F3759DF)

_mesh = plsc.VectorSubcoreMesh(core_axis_name="c", subcore_axis_name="s")
_params = pltpu.CompilerParams(use_tc_tiling_on_sc=False)


def _rsqrt16(d):
    ds_ = jnp.maximum(d, 1.0)
    i = lax.bitcast_convert_type(ds_, jnp.int32)
    i = MAGIC - lax.shift_right_logical(i, 1)
    r = lax.bitcast_convert_type(i, jnp.float32)
    for _ in range(3):
        r = r * (1.5 - 0.5 * ds_ * r * r)
    return jnp.where(d > 0.0, r, 0.0)


_GATHER_DNUMS = lax.GatherDimensionNumbers(
    offset_dims=(), collapsed_slice_dims=(0,), start_index_map=(0,))


def _bcast_lane(v16, j):
    idx = jnp.full((16, 1), j, dtype=jnp.int32)
    return lax.gather(v16, idx, _GATHER_DNUMS, slice_sizes=(1,),
                      mode=lax.GatherScatterMode.PROMISE_IN_BOUNDS)


def _strided_count(s, total):
    return (total - s + 15) // 16


def _fill_zero_rows(buf, nrows):
    z = jnp.zeros((16,), jnp.float32)

    def body(r, _):
        for k in range(D // 16):
            buf[r, pl.ds(16 * k, 16)] = z
        return 0

    lax.fori_loop(0, nrows, body, 0)


@functools.partial(
    pl.kernel,
    mesh=_mesh,
    compiler_params=_params,
    out_type=(
        jax.ShapeDtypeStruct((NP,), jnp.float32),
        jax.ShapeDtypeStruct((NP, D), jnp.float32),
        jax.ShapeDtypeStruct((E,), jnp.int32),
        jax.ShapeDtypeStruct((E,), jnp.int32),
        jax.ShapeDtypeStruct((E,), jnp.int32),
    ),
    scratch_types=[
        pltpu.VMEM_SHARED((S_ROWS,), jnp.float32),
        pltpu.VMEM((1, EB), jnp.int32),
        pltpu.VMEM((1, EB), jnp.int32),
        pltpu.VMEM((1, EB), jnp.int32),
        pltpu.VMEM((1, EB), jnp.int32),
        pltpu.VMEM((EB,), jnp.float32),
        pltpu.VMEM((EB,), jnp.float32),
        pltpu.VMEM((EB,), jnp.float32),
        pltpu.VMEM((EB, D), jnp.float32),
    ],
)
def _prep(x0_hbm, row_hbm, col_hbm, dis_hbm, y0_hbm, rl0_hbm, rl1_hbm,
          colp_hbm, deg_sh, rr, rc, lb, cb, ones, fbuf, disb, xb):
    c = lax.axis_index("c")
    s = lax.axis_index("s")
    lanes = lax.iota(jnp.int32, 16)

    for k in range(EB // 16):
        ones[pl.ds(16 * k, 16)] = jnp.ones((16,), jnp.float32)
        fbuf[pl.ds(16 * k, 16)] = jnp.zeros((16,), jnp.float32)

    def zero_body(i, _):
        b = s + 16 * i
        pltpu.sync_copy(fbuf, deg_sh.at[pl.ds(b * 128, 128)])
        return 0

    lax.fori_loop(0, _strided_count(s, NBLK_S), zero_body, 0)
    plsc.subcore_barrier()

    def edge_body(i, _):
        b = s + 16 * i
        off = b * EB
        pltpu.sync_copy(row_hbm.at[pl.ds(off, EB)], rr.at[0])
        pltpu.sync_copy(col_hbm.at[pl.ds(off, EB)], rc.at[0])
        dummy = jnp.full((16,), P, jnp.int32) + lanes + (
            jnp.full((16,), (b % 8) * 16, jnp.int32))
        for k in range(EB // 16):
            r16 = rr[0, pl.ds(16 * k, 16)]
            c16 = rc[0, pl.ds(16 * k, 16)]
            inh = jnp.where(c == 0, r16 < HALF, r16 >= HALF)
            loc = jnp.where(c == 0, r16, r16 - HALF)
            lb[0, pl.ds(16 * k, 16)] = jnp.where(inh, loc, dummy)
            cb[0, pl.ds(16 * k, 16)] = jnp.where(
                c16 >= HALF, c16 + PAD, c16)
        pltpu.sync_copy(ones, deg_sh.at[lb.at[0]], add=True)

        @pl.when(c == 0)
        def _():
            pltpu.sync_copy(lb.at[0], rl0_hbm.at[pl.ds(off, EB)])
            pltpu.sync_copy(cb.at[0], colp_hbm.at[pl.ds(off, EB)])

        @pl.when(c == 1)
        def _():
            pltpu.sync_copy(lb.at[0], rl1_hbm.at[pl.ds(off, EB)])

        return 0

    lax.fori_loop(0, _strided_count(s, NBLK_E), edge_body, 0)
    plsc.subcore_barrier()

    def node_body(i, _):
        b = s + 16 * i
        off = b * 128
        g = c * P + off
        pltpu.sync_copy(deg_sh.at[pl.ds(off, 128)], fbuf)
        for k in range(128 // 16):
            disb[pl.ds(16 * k, 16)] = _rsqrt16(fbuf[pl.ds(16 * k, 16)])
        pltpu.sync_copy(disb, dis_hbm.at[pl.ds(g, 128)])
        pltpu.sync_copy(x0_hbm.at[pl.ds(g, 128)], xb)

        def row_body(j8, _):
            dis16 = disb[pl.ds(16 * j8, 16)]
            for jj in range(16):
                dv = _bcast_lane(dis16, jj)
                r = 16 * j8 + jj
                for k in range(D // 16):
                    xb[r, pl.ds(16 * k, 16)] = (
                        xb[r, pl.ds(16 * k, 16)] * dv)
            return 0

        lax.fori_loop(0, 8, row_body, 0)
        pltpu.sync_copy(xb, y0_hbm.at[pl.ds(g, 128)])
        return 0

    lax.fori_loop(0, _strided_count(s, NBLK_N), node_body, 0)


def _make_layer(last):
    out_type = (jax.ShapeDtypeStruct((NP, D), jnp.float32),)
    if not last:
        out_type = out_type + (jax.ShapeDtypeStruct((NP, D), jnp.float32),)

    @functools.partial(
        pl.kernel,
        mesh=_mesh,
        compiler_params=_params,
        out_type=out_type,
        scratch_types=[
            pltpu.VMEM_SHARED((S_ROWS, D), jnp.float32),
            pltpu.VMEM((1, EB), jnp.int32),
            pltpu.VMEM((1, EB), jnp.int32),
            pltpu.VMEM((EB, D), jnp.float32),
            pltpu.VMEM((EB, D), jnp.float32),
            pltpu.VMEM((EB,), jnp.float32),
            pltpu.SemaphoreType.DMA,
        ],
    )
    def layer(y_hbm, dis_hbm, acc_hbm, rl0_hbm, rl1_hbm, colp_hbm,
              *out_and_scratch):
        if last:
            (accout_hbm, s_sh, ib_r, ib_c, rows, accb, disb,
             sem) = out_and_scratch
            yout_hbm = None
        else:
            (accout_hbm, yout_hbm, s_sh, ib_r, ib_c, rows, accb, disb,
             sem) = out_and_scratch
        c = lax.axis_index("c")
        s = lax.axis_index("s")

        _fill_zero_rows(rows, EB)

        def zero_body(i, _):
            b = s + 16 * i
            pltpu.sync_copy(rows, s_sh.at[pl.ds(b * 128, 128)])
            return 0

        lax.fori_loop(0, _strided_count(s, NBLK_S), zero_body, 0)
        plsc.subcore_barrier()

        def edge_body(i, _):
            b = s + 16 * i
            off = b * EB

            @pl.when(c == 0)
            def _():
                pltpu.sync_copy(rl0_hbm.at[pl.ds(off, EB)], ib_r.at[0])

            @pl.when(c == 1)
            def _():
                pltpu.sync_copy(rl1_hbm.at[pl.ds(off, EB)], ib_r.at[0])

            pltpu.sync_copy(colp_hbm.at[pl.ds(off, EB)], ib_c.at[0])
            pltpu.async_copy(y_hbm.at[ib_c.at[0]], rows, sem).wait()
            pltpu.sync_copy(rows, s_sh.at[ib_r.at[0]], add=True)
            return 0

        lax.fori_loop(0, _strided_count(s, NBLK_E), edge_body, 0)
        plsc.subcore_barrier()

        def node_body(i, _):
            b = s + 16 * i
            off = b * 128
            g = c * P + off
            pltpu.sync_copy(s_sh.at[pl.ds(off, 128)], rows)
            pltpu.sync_copy(dis_hbm.at[pl.ds(g, 128)], disb)
            pltpu.sync_copy(acc_hbm.at[pl.ds(g, 128)], accb)

            def row_body(j8, _):
                dis16 = disb[pl.ds(16 * j8, 16)]
                for jj in range(16):
                    dv = _bcast_lane(dis16, jj)
                    r = 16 * j8 + jj
                    for k in range(D // 16):
                        sl = pl.ds(16 * k, 16)
                        x = rows[r, sl] * dv
                        a = accb[r, sl] + x
                        if last:
                            accb[r, sl] = a * 0.25
                        else:
                            accb[r, sl] = a
                            rows[r, sl] = x * dv
                return 0

            lax.fori_loop(0, 8, row_body, 0)
            pltpu.sync_copy(accb, accout_hbm.at[pl.ds(g, 128)])
            if not last:
                pltpu.sync_copy(rows, yout_hbm.at[pl.ds(g, 128)])
            return 0

        lax.fori_loop(0, _strided_count(s, NBLK_N), node_body, 0)

    return layer


_layer_mid = _make_layer(last=False)
_layer_last = _make_layer(last=True)


def kernel(user_emb, item_emb, edge_index):
    zpad = jnp.zeros((PAD, D), jnp.float32)
    x0 = jnp.concatenate([user_emb, zpad, item_emb, zpad], axis=0)
    row = edge_index[0]
    col = edge_index[1]
    dis, y0, rl0, rl1, colp = _prep(x0, row, col)
    acc1, y1 = _layer_mid(y0, dis, x0, rl0, rl1, colp)
    acc2, y2 = _layer_mid(y1, dis, acc1, rl0, rl1, colp)
    (fin,) = _layer_last(y2, dis, acc2, rl0, rl1, colp)
    return (fin[:N_USERS], fin[P:P + N_ITEMS])
